# baseline (speedup 1.0000x reference)
import functools

import jax
import jax.numpy as jnp
from jax import lax
from jax.experimental import pallas as pl
from jax.experimental.pallas import tpu as pltpu
from jax.experimental.pallas import tpu_sc as plsc

_N = 10000
_P = 512
_C = 256
_A = 128
_NP = _N + _P
_RPAD = 10752
_B1 = 512
_B4 = 1024
_N4 = 10240
_ACC = 10624
_E = 160000
_EP = 163840
_NCH = 80
_NW = 5
_WCH = 16
_ZCH = 83

_f32 = jnp.float32


def _ln(t, g, b):
    m = jnp.mean(t, axis=-1, keepdims=True)
    v = jnp.mean((t - m) ** 2, axis=-1, keepdims=True)
    return (t - m) / jnp.sqrt(v + 1e-5) * g + b


def _elu(t):
    return jnp.where(t > 0, t, jnp.exp(t) - 1.0)


def _k1_body(x_ref, w1t_ref, b1_ref, lg_ref, lb_ref, vadd_ref, wet_ref, be_ref,
             hl_ref, hr_ref, he_ref):
    t = jnp.dot(x_ref[...], w1t_ref[...], preferred_element_type=_f32) + b1_ref[...]
    h = _elu(_ln(t, lg_ref[...], lb_ref[...])) + vadd_ref[...]
    hl_ref[...] = h[:, :128]
    hr_ref[...] = h[:, 128:]
    he_ref[...] = jnp.dot(h, wet_ref[...], preferred_element_type=_f32) + be_ref[...]


def _k3b_body(hev_ref, al_ref, ar_ref, lg2_ref, lb2_ref, wdt_ref, bd_ref, ab_ref,
              vb_ref, wbt_ref, wat_ref, ba_ref, wot_ref, bo_ref,
              d_ref, g_ref, reps_ref):
    zv = _elu(_ln(hev_ref[...] + jnp.concatenate([al_ref[...], ar_ref[...]], 1),
                  lg2_ref[...], lb2_ref[...]))
    d_ref[...] = jnp.dot(zv, wdt_ref[...], preferred_element_type=_f32) + bd_ref[...] + ab_ref[...]
    zv2 = zv + vb_ref[...]
    g = jnp.dot(zv2, wbt_ref[...], preferred_element_type=_f32)
    g_ref[...] = g
    u = _elu(_ln(jnp.dot(zv2, wat_ref[...], preferred_element_type=_f32) + g + ba_ref[...],
                 lg2_ref[...], lb2_ref[...]))
    reps_ref[...] = jnp.dot(u, wot_ref[...], preferred_element_type=_f32) + bo_ref[...]


def _k4_body(he_ref, al_ref, ar_ref, lg2_ref, lb2_ref, wst_ref, bs_ref, d_ref,
             g_ref, wat_ref, ba_ref, wot_ref, bo_ref, y_ref, ci_ref):
    z = _elu(_ln(he_ref[...] + jnp.concatenate([al_ref[...], ar_ref[...]], 1),
                 lg2_ref[...], lb2_ref[...]))
    s = jnp.dot(z, wst_ref[...], preferred_element_type=_f32) + bs_ref[...]
    alpha = lax.dot_general(s, d_ref[...], (((1,), (1,)), ((), ())),
                            preferred_element_type=_f32)
    mx = jnp.max(alpha, axis=1, keepdims=True)
    col = lax.broadcasted_iota(jnp.int32, alpha.shape, 1)
    cidx = jnp.min(jnp.where(alpha >= mx, col, _P), axis=1, keepdims=True)
    oh = (col == cidx).astype(_f32)
    sel = jnp.dot(oh, g_ref[...], preferred_element_type=_f32)
    u = _elu(_ln(jnp.dot(z, wat_ref[...], preferred_element_type=_f32) + sel + ba_ref[...],
                 lg2_ref[...], lb2_ref[...]))
    y_ref[...] = jnp.dot(u, wot_ref[...], preferred_element_type=_f32) + bo_ref[...]
    ci_ref[...] = jnp.broadcast_to(cidx, (cidx.shape[0], 128))


def _full(shape):
    return pl.BlockSpec(shape, lambda i: (0, 0))


_sc_mesh = plsc.VectorSubcoreMesh(
    core_axis_name="c", subcore_axis_name="s", num_cores=2, num_subcores=16)


@functools.partial(
    pl.kernel,
    out_type=[
        jax.ShapeDtypeStruct((_ACC, 128), _f32),
        jax.ShapeDtypeStruct((_ACC, 128), _f32),
    ],
    mesh=_sc_mesh,
    scratch_types=[
        pltpu.VMEM((_WCH, 128), jnp.int32),
        pltpu.VMEM((_WCH, 128), jnp.int32),
        pltpu.VMEM((128, 128), _f32),
        pltpu.VMEM((128, 128), _f32),
        pltpu.VMEM_SHARED((_ACC, 128), _f32),
        pltpu.SemaphoreType.DMA,
        pltpu.SemaphoreType.DMA,
        pltpu.SemaphoreType.DMA,
        pltpu.SemaphoreType.DMA,
    ],
)
def _k2(hl_ref, hr_ref, src_ref, dst_ref, zeros_ref, out0_ref, out1_ref,
        src_v, dst_v, rows0, rows1, acc, sem_g0, sem_g1, sem_s0, sem_s1):
    c = lax.axis_index("c")
    s = lax.axis_index("s")

    pltpu.sync_copy(zeros_ref, rows0)

    def zbody(j, carry):
        k = j * 16 + s

        @pl.when(k < _ZCH)
        def _():
            pltpu.sync_copy(rows0, acc.at[pl.ds(k * 128, 128)])
        return carry

    lax.fori_loop(0, 6, zbody, 0)
    plsc.subcore_barrier()

    def _edge_loop(h_ref):
        def window(w, carry):
            base = s * _NCH + w * _WCH
            pltpu.sync_copy(src_ref.at[pl.ds(base, _WCH)], src_v)
            pltpu.sync_copy(dst_ref.at[pl.ds(base, _WCH)], dst_v)
            pltpu.async_copy(h_ref.at[src_v.at[0]], rows0, sem_g0)
            pltpu.async_copy(h_ref.at[src_v.at[1]], rows1, sem_g1)

            def pair(i, carry2):
                j0 = 2 * i
                j1 = j0 + 1
                pltpu.make_async_copy(h_ref.at[src_v.at[j0]], rows0, sem_g0).wait()
                pltpu.async_copy(rows0, acc.at[dst_v.at[j0]], sem_s0, add=True)
                pltpu.make_async_copy(h_ref.at[src_v.at[j1]], rows1, sem_g1).wait()
                pltpu.async_copy(rows1, acc.at[dst_v.at[j1]], sem_s1, add=True)

                pltpu.make_async_copy(rows0, acc.at[dst_v.at[j0]], sem_s0).wait()

                @pl.when(j0 + 2 < _WCH)
                def _():
                    pltpu.async_copy(h_ref.at[src_v.at[j0 + 2]], rows0, sem_g0)

                pltpu.make_async_copy(rows1, acc.at[dst_v.at[j1]], sem_s1).wait()

                @pl.when(j1 + 2 < _WCH)
                def _():
                    pltpu.async_copy(h_ref.at[src_v.at[j1 + 2]], rows1, sem_g1)
                return carry2

            lax.fori_loop(0, _WCH // 2, pair, 0)
            return carry

        lax.fori_loop(0, _NW, window, 0)

    @pl.when(c == 0)
    def _():
        _edge_loop(hl_ref)

    @pl.when(c == 1)
    def _():
        _edge_loop(hr_ref)

    plsc.subcore_barrier()

    def wbody(j, carry):
        k = j * 16 + s

        @pl.when(k < _ZCH)
        def _():
            pltpu.sync_copy(acc.at[pl.ds(k * 128, 128)], rows0)

            @pl.when(c == 0)
            def _():
                pltpu.sync_copy(rows0, out0_ref.at[pl.ds(k * 128, 128)])

            @pl.when(c == 1)
            def _():
                pltpu.sync_copy(rows0, out1_ref.at[pl.ds(k * 128, 128)])
        return carry

    lax.fori_loop(0, 6, wbody, 0)


def kernel(x, adjs, mapping, edge_mask, params):
    p = params

    xc = jnp.concatenate([x[:_N], p['vnode_embed']], axis=0)
    xp = jnp.pad(xc, ((0, _RPAD - _NP), (0, 0)))
    vadd = jnp.pad(p['vb_hid'], ((_N, _RPAD - _NP), (0, 0)))
    w1t = p['W_in2hid'].T
    wet = p['W_enc'].T
    b1 = p['b_in2hid'][None]
    be = p['b_enc'][None]
    lg = p['ln_hid_g'][None]
    lb = p['ln_hid_b'][None]
    lg2 = p['ln_enc_g'][None]
    lb2 = p['ln_enc_b'][None]
    wst = jnp.pad(p['Ws'].T, ((0, 0), (0, _A - 64)))
    bs = jnp.pad(p['bs'][None], ((0, 0), (0, _A - 64)))
    wdt = jnp.pad(p['Wd'].T, ((0, 0), (0, _A - 64)))
    bd = jnp.pad(p['bd'][None], ((0, 0), (0, _A - 64)))
    ab = jnp.pad(p['attn_bias'], ((0, 0), (0, _A - 64)))
    wat = p['W_aggr'][:, :_C].T
    wbt = p['W_aggr'][:, _C:].T
    ba = p['b_aggr'][None]
    wot = p['W_out'].T
    bo = p['b_out'][None]

    nb1 = _RPAD // _B1
    hl, hr, he = pl.pallas_call(
        _k1_body,
        grid=(nb1,),
        in_specs=[
            pl.BlockSpec((_B1, _C), lambda i: (i, 0)),
            _full((_C, _C)), _full((1, _C)), _full((1, _C)), _full((1, _C)),
            pl.BlockSpec((_B1, _C), lambda i: (i, 0)),
            _full((_C, _C)), _full((1, _C)),
        ],
        out_specs=[
            pl.BlockSpec((_B1, 128), lambda i: (i, 0)),
            pl.BlockSpec((_B1, 128), lambda i: (i, 0)),
            pl.BlockSpec((_B1, _C), lambda i: (i, 0)),
        ],
        out_shape=[
            jax.ShapeDtypeStruct((_RPAD, 128), _f32),
            jax.ShapeDtypeStruct((_RPAD, 128), _f32),
            jax.ShapeDtypeStruct((_RPAD, _C), _f32),
        ],
    )(xp, w1t, b1, lg, lb, vadd, wet, be)

    srcm = jnp.pad(adjs[0, 0], (0, _EP - _E)).reshape(-1, 128)
    dstm = jnp.pad(jnp.where(edge_mask, adjs[0, 1], _NP), (0, _EP - _E),
                   constant_values=_NP).reshape(-1, 128)
    zeros = jnp.zeros((128, 128), _f32)
    al_full, ar_full = _k2(hl, hr, srcm, dstm, zeros)

    d, g, reps = pl.pallas_call(
        _k3b_body,
        grid=(1,),
        in_specs=[
            _full((_P, _C)), _full((_P, 128)), _full((_P, 128)),
            _full((1, _C)), _full((1, _C)),
            _full((_C, _A)), _full((1, _A)), _full((_P, _A)),
            _full((_P, _C)), _full((_C, _C)), _full((_C, _C)), _full((1, _C)),
            _full((_C, _C)), _full((1, _C)),
        ],
        out_specs=[_full((_P, _A)), _full((_P, _C)), _full((_P, _C))],
        out_shape=[
            jax.ShapeDtypeStruct((_P, _A), _f32),
            jax.ShapeDtypeStruct((_P, _C), _f32),
            jax.ShapeDtypeStruct((_P, _C), _f32),
        ],
    )(he[_N:_NP], al_full[_N:_NP], ar_full[_N:_NP], lg2, lb2, wdt, bd, ab,
      p['vb_dcd'], wbt, wat, ba, wot, bo)

    nb4 = _N4 // _B4
    y, ci = pl.pallas_call(
        _k4_body,
        grid=(nb4,),
        in_specs=[
            pl.BlockSpec((_B4, _C), lambda i: (i, 0)),
            pl.BlockSpec((_B4, 128), lambda i: (i, 0)),
            pl.BlockSpec((_B4, 128), lambda i: (i, 0)),
            _full((1, _C)), _full((1, _C)),
            _full((_C, _A)), _full((1, _A)),
            _full((_P, _A)), _full((_P, _C)),
            _full((_C, _C)), _full((1, _C)), _full((_C, _C)), _full((1, _C)),
        ],
        out_specs=[
            pl.BlockSpec((_B4, _C), lambda i: (i, 0)),
            pl.BlockSpec((_B4, 128), lambda i: (i, 0)),
        ],
        out_shape=[
            jax.ShapeDtypeStruct((_N4, _C), _f32),
            jax.ShapeDtypeStruct((_N4, 128), jnp.int32),
        ],
    )(he[:_N4], al_full[:_N4], ar_full[:_N4], lg2, lb2, wst, bs, d, g,
      wat, ba, wot, bo)

    out = y[:_N]
    cluster_mapping = ci[:_N, 0]
    loss = jnp.float32(0.0)
    return out, loss, reps, cluster_mapping


# speedup vs baseline: 1.1169x; 1.1169x over previous
import functools

import jax
import jax.numpy as jnp
from jax import lax
from jax.experimental import pallas as pl
from jax.experimental.pallas import tpu as pltpu
from jax.experimental.pallas import tpu_sc as plsc

_N = 10000
_P = 512
_C = 256
_A = 128
_NP = _N + _P
_RPAD = 10752
_B1 = 512
_B4 = 1024
_N4 = 10240
_ACC = 10624
_E = 160000
_EP = 163840
_NCH = 80
_NW = 5
_WCH = 16
_ZCH = 83

_f32 = jnp.float32


def _ln(t, g, b):
    m = jnp.mean(t, axis=-1, keepdims=True)
    v = jnp.mean((t - m) ** 2, axis=-1, keepdims=True)
    return (t - m) / jnp.sqrt(v + 1e-5) * g + b


def _elu(t):
    return jnp.where(t > 0, t, jnp.exp(t) - 1.0)


def _k1_body(x_ref, vnp_ref, vbp_ref, w1t_ref, b1_ref, lg_ref, lb_ref,
             wet_ref, be_ref, hl_ref, hr_ref, he_ref):
    base = pl.program_id(0) * _B1
    o = pl.multiple_of(jnp.clip(base - _N + _P, 0, 2 * _P), 16)
    rowid = base + lax.broadcasted_iota(jnp.int32, (_B1, 1), 0)
    vn = vnp_ref[pl.ds(o, _B1), :]
    xin = jnp.where(rowid >= _N, vn, x_ref[...])
    t = jnp.dot(xin, w1t_ref[...], preferred_element_type=_f32) + b1_ref[...]
    h = _elu(_ln(t, lg_ref[...], lb_ref[...])) + vbp_ref[pl.ds(o, _B1), :]
    hl_ref[...] = h[:, :128]
    hr_ref[...] = h[:, 128:]
    he_ref[...] = jnp.dot(h, wet_ref[...], preferred_element_type=_f32) + be_ref[...]


def _k3b_body(hev_ref, al_ref, ar_ref, lg2_ref, lb2_ref, wdt_ref, bd_ref, ab_ref,
              vb_ref, wbt_ref, wat_ref, ba_ref, wot_ref, bo_ref,
              d_ref, g_ref, reps_ref):
    zv = _elu(_ln(hev_ref[...] + jnp.concatenate([al_ref[...], ar_ref[...]], 1),
                  lg2_ref[...], lb2_ref[...]))
    d_ref[...] = jnp.dot(zv, wdt_ref[...], preferred_element_type=_f32) + bd_ref[...] + ab_ref[...]
    zv2 = zv + vb_ref[...]
    g = jnp.dot(zv2, wbt_ref[...], preferred_element_type=_f32)
    g_ref[...] = g
    u = _elu(_ln(jnp.dot(zv2, wat_ref[...], preferred_element_type=_f32) + g + ba_ref[...],
                 lg2_ref[...], lb2_ref[...]))
    reps_ref[...] = jnp.dot(u, wot_ref[...], preferred_element_type=_f32) + bo_ref[...]


def _k4_body(he_ref, al_ref, ar_ref, lg2_ref, lb2_ref, wst_ref, bs_ref, d_ref,
             g_ref, wat_ref, ba_ref, wot_ref, bo_ref, y_ref, ci_ref):
    z = _elu(_ln(he_ref[...] + jnp.concatenate([al_ref[...], ar_ref[...]], 1),
                 lg2_ref[...], lb2_ref[...]))
    s = jnp.dot(z, wst_ref[...], preferred_element_type=_f32) + bs_ref[...]
    alpha = lax.dot_general(s, d_ref[...], (((1,), (1,)), ((), ())),
                            preferred_element_type=_f32)
    mx = jnp.max(alpha, axis=1, keepdims=True)
    col = lax.broadcasted_iota(jnp.int32, alpha.shape, 1)
    cidx = jnp.min(jnp.where(alpha >= mx, col, _P), axis=1, keepdims=True)
    oh = (col == cidx).astype(_f32)
    sel = jnp.dot(oh, g_ref[...], preferred_element_type=_f32)
    u = _elu(_ln(jnp.dot(z, wat_ref[...], preferred_element_type=_f32) + sel + ba_ref[...],
                 lg2_ref[...], lb2_ref[...]))
    y_ref[...] = jnp.dot(u, wot_ref[...], preferred_element_type=_f32) + bo_ref[...]
    ci_ref[...] = jnp.broadcast_to(cidx, (cidx.shape[0], 128))


def _full(shape):
    return pl.BlockSpec(shape, lambda i: (0, 0))


_sc_mesh = plsc.VectorSubcoreMesh(
    core_axis_name="c", subcore_axis_name="s", num_cores=2, num_subcores=16)


@functools.partial(
    pl.kernel,
    out_type=[
        jax.ShapeDtypeStruct((_ACC, 128), _f32),
        jax.ShapeDtypeStruct((_ACC, 128), _f32),
    ],
    mesh=_sc_mesh,
    scratch_types=[
        pltpu.VMEM((_WCH, 128), jnp.int32),
        pltpu.VMEM((_WCH, 128), jnp.int32),
        pltpu.VMEM((128, 128), _f32),
        pltpu.VMEM((128, 128), _f32),
        pltpu.VMEM_SHARED((_ACC, 128), _f32),
        pltpu.SemaphoreType.DMA,
        pltpu.SemaphoreType.DMA,
        pltpu.SemaphoreType.DMA,
        pltpu.SemaphoreType.DMA,
    ],
)
def _k2(hl_ref, hr_ref, src_ref, dst_ref, zeros_ref, out0_ref, out1_ref,
        src_v, dst_v, rows0, rows1, acc, sem_g0, sem_g1, sem_s0, sem_s1):
    c = lax.axis_index("c")
    s = lax.axis_index("s")

    pltpu.sync_copy(zeros_ref, rows0)

    def zbody(j, carry):
        k = j * 16 + s

        @pl.when(k < _ZCH)
        def _():
            pltpu.sync_copy(rows0, acc.at[pl.ds(k * 128, 128)])
        return carry

    lax.fori_loop(0, 6, zbody, 0)
    plsc.subcore_barrier()

    def _edge_loop(h_ref):
        def window(w, carry):
            base = s * _NCH + w * _WCH
            pltpu.sync_copy(src_ref.at[pl.ds(base, _WCH)], src_v)
            pltpu.sync_copy(dst_ref.at[pl.ds(base, _WCH)], dst_v)
            pltpu.async_copy(h_ref.at[src_v.at[0]], rows0, sem_g0)

            def pair(i, carry2):
                j0 = 2 * i
                j1 = j0 + 1
                pltpu.async_copy(h_ref.at[src_v.at[j1]], rows1, sem_g1)
                pltpu.make_async_copy(h_ref.at[src_v.at[j0]], rows0, sem_g0).wait()
                pltpu.sync_copy(rows0, acc.at[dst_v.at[j0]], add=True)

                @pl.when(j1 + 1 < _WCH)
                def _():
                    pltpu.async_copy(h_ref.at[src_v.at[j1 + 1]], rows0, sem_g0)

                pltpu.make_async_copy(h_ref.at[src_v.at[j1]], rows1, sem_g1).wait()
                pltpu.sync_copy(rows1, acc.at[dst_v.at[j1]], add=True)
                return carry2

            lax.fori_loop(0, _WCH // 2, pair, 0)
            return carry

        lax.fori_loop(0, _NW, window, 0)

    @pl.when(c == 0)
    def _():
        _edge_loop(hl_ref)

    @pl.when(c == 1)
    def _():
        _edge_loop(hr_ref)

    plsc.subcore_barrier()

    def wbody(j, carry):
        k = j * 16 + s

        @pl.when(k < _ZCH)
        def _():
            pltpu.sync_copy(acc.at[pl.ds(k * 128, 128)], rows0)

            @pl.when(c == 0)
            def _():
                pltpu.sync_copy(rows0, out0_ref.at[pl.ds(k * 128, 128)])

            @pl.when(c == 1)
            def _():
                pltpu.sync_copy(rows0, out1_ref.at[pl.ds(k * 128, 128)])
        return carry

    lax.fori_loop(0, 6, wbody, 0)


def kernel(x, adjs, mapping, edge_mask, params):
    p = params

    vnp = jnp.pad(p['vnode_embed'], ((_P, _P), (0, 0)))
    vbp = jnp.pad(p['vb_hid'], ((_P, _P), (0, 0)))
    w1t = p['W_in2hid'].T
    wet = p['W_enc'].T
    b1 = p['b_in2hid'][None]
    be = p['b_enc'][None]
    lg = p['ln_hid_g'][None]
    lb = p['ln_hid_b'][None]
    lg2 = p['ln_enc_g'][None]
    lb2 = p['ln_enc_b'][None]
    wst = jnp.pad(p['Ws'].T, ((0, 0), (0, _A - 64)))
    bs = jnp.pad(p['bs'][None], ((0, 0), (0, _A - 64)))
    wdt = jnp.pad(p['Wd'].T, ((0, 0), (0, _A - 64)))
    bd = jnp.pad(p['bd'][None], ((0, 0), (0, _A - 64)))
    ab = jnp.pad(p['attn_bias'], ((0, 0), (0, _A - 64)))
    wat = p['W_aggr'][:, :_C].T
    wbt = p['W_aggr'][:, _C:].T
    ba = p['b_aggr'][None]
    wot = p['W_out'].T
    bo = p['b_out'][None]

    nb1 = pl.cdiv(_NP, _B1)
    hl, hr, he = pl.pallas_call(
        _k1_body,
        grid=(nb1,),
        in_specs=[
            pl.BlockSpec((_B1, _C), lambda i: (i, 0)),
            _full((3 * _P, _C)), _full((3 * _P, _C)),
            _full((_C, _C)), _full((1, _C)), _full((1, _C)), _full((1, _C)),
            _full((_C, _C)), _full((1, _C)),
        ],
        out_specs=[
            pl.BlockSpec((_B1, 128), lambda i: (i, 0)),
            pl.BlockSpec((_B1, 128), lambda i: (i, 0)),
            pl.BlockSpec((_B1, _C), lambda i: (i, 0)),
        ],
        out_shape=[
            jax.ShapeDtypeStruct((_NP, 128), _f32),
            jax.ShapeDtypeStruct((_NP, 128), _f32),
            jax.ShapeDtypeStruct((_NP, _C), _f32),
        ],
    )(x, vnp, vbp, w1t, b1, lg, lb, wet, be)

    srcm = jnp.pad(adjs[0, 0], (0, _EP - _E)).reshape(-1, 128)
    dstm = jnp.pad(jnp.where(edge_mask, adjs[0, 1], _NP), (0, _EP - _E),
                   constant_values=_NP).reshape(-1, 128)
    zeros = jnp.zeros((128, 128), _f32)
    al_full, ar_full = _k2(hl, hr, srcm, dstm, zeros)

    d, g, reps = pl.pallas_call(
        _k3b_body,
        grid=(1,),
        in_specs=[
            _full((_P, _C)), _full((_P, 128)), _full((_P, 128)),
            _full((1, _C)), _full((1, _C)),
            _full((_C, _A)), _full((1, _A)), _full((_P, _A)),
            _full((_P, _C)), _full((_C, _C)), _full((_C, _C)), _full((1, _C)),
            _full((_C, _C)), _full((1, _C)),
        ],
        out_specs=[_full((_P, _A)), _full((_P, _C)), _full((_P, _C))],
        out_shape=[
            jax.ShapeDtypeStruct((_P, _A), _f32),
            jax.ShapeDtypeStruct((_P, _C), _f32),
            jax.ShapeDtypeStruct((_P, _C), _f32),
        ],
    )(he[_N:_NP], al_full[_N:_NP], ar_full[_N:_NP], lg2, lb2, wdt, bd, ab,
      p['vb_dcd'], wbt, wat, ba, wot, bo)

    nb4 = pl.cdiv(_N, _B4)
    y, ci = pl.pallas_call(
        _k4_body,
        grid=(nb4,),
        in_specs=[
            pl.BlockSpec((_B4, _C), lambda i: (i, 0)),
            pl.BlockSpec((_B4, 128), lambda i: (i, 0)),
            pl.BlockSpec((_B4, 128), lambda i: (i, 0)),
            _full((1, _C)), _full((1, _C)),
            _full((_C, _A)), _full((1, _A)),
            _full((_P, _A)), _full((_P, _C)),
            _full((_C, _C)), _full((1, _C)), _full((_C, _C)), _full((1, _C)),
        ],
        out_specs=[
            pl.BlockSpec((_B4, _C), lambda i: (i, 0)),
            pl.BlockSpec((_B4, 128), lambda i: (i, 0)),
        ],
        out_shape=[
            jax.ShapeDtypeStruct((_N, _C), _f32),
            jax.ShapeDtypeStruct((_N, 128), jnp.int32),
        ],
    )(he, al_full, ar_full, lg2, lb2, wst, bs, d, g,
      wat, ba, wot, bo)

    cluster_mapping = ci[:, 0]
    loss = jnp.float32(0.0)
    return y, loss, reps, cluster_mapping


# speedup vs baseline: 1.1957x; 1.0706x over previous
import functools

import jax
import jax.numpy as jnp
from jax import lax
from jax.experimental import pallas as pl
from jax.experimental.pallas import tpu as pltpu
from jax.experimental.pallas import tpu_sc as plsc

_N = 10000
_P = 512
_C = 256
_A = 128
_NP = _N + _P
_RPAD = 10752
_B1 = 512
_B4 = 1024
_N4 = 10240
_ACC = 10624
_E = 160000
_EP = 163840
_NCH = 80
_NW = 5
_WCH = 16
_ZCH = 83

_f32 = jnp.float32


def _ln(t, g, b):
    m = jnp.mean(t, axis=-1, keepdims=True)
    v = jnp.mean((t - m) ** 2, axis=-1, keepdims=True)
    return (t - m) / jnp.sqrt(v + 1e-5) * g + b


def _elu(t):
    return jnp.where(t > 0, t, jnp.exp(t) - 1.0)


def _k1_body(x_ref, vnp_ref, vbp_ref, w1t_ref, b1_ref, lg_ref, lb_ref,
             hl_ref, hr_ref):
    base = pl.program_id(0) * _B1
    o = pl.multiple_of(jnp.clip(base - _N + _P, 0, 2 * _P), 16)
    rowid = base + lax.broadcasted_iota(jnp.int32, (_B1, 1), 0)
    vn = vnp_ref[pl.ds(o, _B1), :]
    xin = jnp.where(rowid >= _N, vn, x_ref[...])
    t = jnp.dot(xin, w1t_ref[...], preferred_element_type=_f32) + b1_ref[...]
    h = _elu(_ln(t, lg_ref[...], lb_ref[...])) + vbp_ref[pl.ds(o, _B1), :]
    hl_ref[...] = h[:, :128]
    hr_ref[...] = h[:, 128:]


def _k1b_body(hl_ref, hr_ref, wet_ref, be_ref, he_ref):
    h = jnp.concatenate([hl_ref[...], hr_ref[...]], 1)
    he_ref[...] = jnp.dot(h, wet_ref[...], preferred_element_type=_f32) + be_ref[...]


def _k3b_body(hev_ref, al_ref, ar_ref, lg2_ref, lb2_ref, wdt_ref, bd_ref, ab_ref,
              vb_ref, wbt_ref, wat_ref, ba_ref, wot_ref, bo_ref,
              d_ref, g_ref, reps_ref):
    zv = _elu(_ln(hev_ref[...] + jnp.concatenate([al_ref[...], ar_ref[...]], 1),
                  lg2_ref[...], lb2_ref[...]))
    d_ref[...] = jnp.dot(zv, wdt_ref[...], preferred_element_type=_f32) + bd_ref[...] + ab_ref[...]
    zv2 = zv + vb_ref[...]
    g = jnp.dot(zv2, wbt_ref[...], preferred_element_type=_f32)
    g_ref[...] = g
    u = _elu(_ln(jnp.dot(zv2, wat_ref[...], preferred_element_type=_f32) + g + ba_ref[...],
                 lg2_ref[...], lb2_ref[...]))
    reps_ref[...] = jnp.dot(u, wot_ref[...], preferred_element_type=_f32) + bo_ref[...]


def _k4_body(he_ref, al_ref, ar_ref, lg2_ref, lb2_ref, wst_ref, bs_ref, d_ref,
             g_ref, wat_ref, ba_ref, wot_ref, bo_ref, y_ref, ci_ref):
    z = _elu(_ln(he_ref[...] + jnp.concatenate([al_ref[...], ar_ref[...]], 1),
                 lg2_ref[...], lb2_ref[...]))
    s = jnp.dot(z, wst_ref[...], preferred_element_type=_f32) + bs_ref[...]
    alpha = lax.dot_general(s, d_ref[...], (((1,), (1,)), ((), ())),
                            preferred_element_type=_f32)
    mx = jnp.max(alpha, axis=1, keepdims=True)
    col = lax.broadcasted_iota(jnp.int32, alpha.shape, 1)
    cidx = jnp.min(jnp.where(alpha >= mx, col, _P), axis=1, keepdims=True)
    oh = (col == cidx).astype(_f32)
    sel = jnp.dot(oh, g_ref[...], preferred_element_type=_f32)
    u = _elu(_ln(jnp.dot(z, wat_ref[...], preferred_element_type=_f32) + sel + ba_ref[...],
                 lg2_ref[...], lb2_ref[...]))
    y_ref[...] = jnp.dot(u, wot_ref[...], preferred_element_type=_f32) + bo_ref[...]
    ci_ref[...] = jnp.broadcast_to(cidx, (cidx.shape[0], 128))


def _full(shape):
    return pl.BlockSpec(shape, lambda i: (0, 0))


_sc_mesh = plsc.VectorSubcoreMesh(
    core_axis_name="c", subcore_axis_name="s", num_cores=2, num_subcores=16)


@functools.partial(
    pl.kernel,
    out_type=[
        jax.ShapeDtypeStruct((_ACC, 128), _f32),
        jax.ShapeDtypeStruct((_ACC, 128), _f32),
    ],
    mesh=_sc_mesh,
    scratch_types=[
        pltpu.VMEM((_WCH, 128), jnp.int32),
        pltpu.VMEM((_WCH, 128), jnp.int32),
        pltpu.VMEM((128, 128), _f32),
        pltpu.VMEM((128, 128), _f32),
        pltpu.VMEM_SHARED((_ACC, 128), _f32),
        pltpu.SemaphoreType.DMA,
        pltpu.SemaphoreType.DMA,
        pltpu.SemaphoreType.DMA,
        pltpu.SemaphoreType.DMA,
    ],
)
def _k2(hl_ref, hr_ref, src_ref, dst_ref, zeros_ref, out0_ref, out1_ref,
        src_v, dst_v, rows0, rows1, acc, sem_g0, sem_g1, sem_s0, sem_s1):
    c = lax.axis_index("c")
    s = lax.axis_index("s")

    pltpu.sync_copy(zeros_ref, rows0)

    def zbody(j, carry):
        k = j * 16 + s

        @pl.when(k < _ZCH)
        def _():
            pltpu.sync_copy(rows0, acc.at[pl.ds(k * 128, 128)])
        return carry

    lax.fori_loop(0, 6, zbody, 0)
    plsc.subcore_barrier()

    def _edge_loop(h_ref):
        def window(w, carry):
            base = s * _NCH + w * _WCH
            pltpu.sync_copy(src_ref.at[pl.ds(base, _WCH)], src_v)
            pltpu.sync_copy(dst_ref.at[pl.ds(base, _WCH)], dst_v)
            pltpu.async_copy(h_ref.at[src_v.at[0]], rows0, sem_g0)

            def pair(i, carry2):
                j0 = 2 * i
                j1 = j0 + 1
                pltpu.async_copy(h_ref.at[src_v.at[j1]], rows1, sem_g1)
                pltpu.make_async_copy(h_ref.at[src_v.at[j0]], rows0, sem_g0).wait()
                pltpu.sync_copy(rows0, acc.at[dst_v.at[j0]], add=True)

                @pl.when(j1 + 1 < _WCH)
                def _():
                    pltpu.async_copy(h_ref.at[src_v.at[j1 + 1]], rows0, sem_g0)

                pltpu.make_async_copy(h_ref.at[src_v.at[j1]], rows1, sem_g1).wait()
                pltpu.sync_copy(rows1, acc.at[dst_v.at[j1]], add=True)
                return carry2

            lax.fori_loop(0, _WCH // 2, pair, 0)
            return carry

        lax.fori_loop(0, _NW, window, 0)

    @pl.when(c == 0)
    def _():
        _edge_loop(hl_ref)

    @pl.when(c == 1)
    def _():
        _edge_loop(hr_ref)

    plsc.subcore_barrier()

    def wbody(j, carry):
        k = j * 16 + s

        @pl.when(k < _ZCH)
        def _():
            pltpu.sync_copy(acc.at[pl.ds(k * 128, 128)], rows0)

            @pl.when(c == 0)
            def _():
                pltpu.sync_copy(rows0, out0_ref.at[pl.ds(k * 128, 128)])

            @pl.when(c == 1)
            def _():
                pltpu.sync_copy(rows0, out1_ref.at[pl.ds(k * 128, 128)])
        return carry

    lax.fori_loop(0, 6, wbody, 0)


def kernel(x, adjs, mapping, edge_mask, params):
    p = params

    vnp = jnp.pad(p['vnode_embed'], ((_P, _P), (0, 0)))
    vbp = jnp.pad(p['vb_hid'], ((_P, _P), (0, 0)))
    w1t = p['W_in2hid'].T
    wet = p['W_enc'].T
    b1 = p['b_in2hid'][None]
    be = p['b_enc'][None]
    lg = p['ln_hid_g'][None]
    lb = p['ln_hid_b'][None]
    lg2 = p['ln_enc_g'][None]
    lb2 = p['ln_enc_b'][None]
    wst = jnp.pad(p['Ws'].T, ((0, 0), (0, _A - 64)))
    bs = jnp.pad(p['bs'][None], ((0, 0), (0, _A - 64)))
    wdt = jnp.pad(p['Wd'].T, ((0, 0), (0, _A - 64)))
    bd = jnp.pad(p['bd'][None], ((0, 0), (0, _A - 64)))
    ab = jnp.pad(p['attn_bias'], ((0, 0), (0, _A - 64)))
    wat = p['W_aggr'][:, :_C].T
    wbt = p['W_aggr'][:, _C:].T
    ba = p['b_aggr'][None]
    wot = p['W_out'].T
    bo = p['b_out'][None]

    nb1 = pl.cdiv(_NP, _B1)
    hl, hr = pl.pallas_call(
        _k1_body,
        grid=(nb1,),
        in_specs=[
            pl.BlockSpec((_B1, _C), lambda i: (i, 0)),
            _full((3 * _P, _C)), _full((3 * _P, _C)),
            _full((_C, _C)), _full((1, _C)), _full((1, _C)), _full((1, _C)),
        ],
        out_specs=[
            pl.BlockSpec((_B1, 128), lambda i: (i, 0)),
            pl.BlockSpec((_B1, 128), lambda i: (i, 0)),
        ],
        out_shape=[
            jax.ShapeDtypeStruct((_NP, 128), _f32),
            jax.ShapeDtypeStruct((_NP, 128), _f32),
        ],
    )(x, vnp, vbp, w1t, b1, lg, lb)

    srcm = jnp.pad(adjs[0, 0], (0, _EP - _E)).reshape(-1, 128)
    dstm = jnp.pad(jnp.where(edge_mask, adjs[0, 1], _NP), (0, _EP - _E),
                   constant_values=_NP).reshape(-1, 128)
    zeros = jnp.zeros((128, 128), _f32)
    al_full, ar_full = _k2(hl, hr, srcm, dstm, zeros)

    he = pl.pallas_call(
        _k1b_body,
        grid=(nb1,),
        in_specs=[
            pl.BlockSpec((_B1, 128), lambda i: (i, 0)),
            pl.BlockSpec((_B1, 128), lambda i: (i, 0)),
            _full((_C, _C)), _full((1, _C)),
        ],
        out_specs=pl.BlockSpec((_B1, _C), lambda i: (i, 0)),
        out_shape=jax.ShapeDtypeStruct((_NP, _C), _f32),
    )(hl, hr, wet, be)

    d, g, reps = pl.pallas_call(
        _k3b_body,
        grid=(1,),
        in_specs=[
            _full((_P, _C)), _full((_P, 128)), _full((_P, 128)),
            _full((1, _C)), _full((1, _C)),
            _full((_C, _A)), _full((1, _A)), _full((_P, _A)),
            _full((_P, _C)), _full((_C, _C)), _full((_C, _C)), _full((1, _C)),
            _full((_C, _C)), _full((1, _C)),
        ],
        out_specs=[_full((_P, _A)), _full((_P, _C)), _full((_P, _C))],
        out_shape=[
            jax.ShapeDtypeStruct((_P, _A), _f32),
            jax.ShapeDtypeStruct((_P, _C), _f32),
            jax.ShapeDtypeStruct((_P, _C), _f32),
        ],
    )(he[_N:_NP], al_full[_N:_NP], ar_full[_N:_NP], lg2, lb2, wdt, bd, ab,
      p['vb_dcd'], wbt, wat, ba, wot, bo)

    nb4 = pl.cdiv(_N, _B4)
    y, ci = pl.pallas_call(
        _k4_body,
        grid=(nb4,),
        in_specs=[
            pl.BlockSpec((_B4, _C), lambda i: (i, 0)),
            pl.BlockSpec((_B4, 128), lambda i: (i, 0)),
            pl.BlockSpec((_B4, 128), lambda i: (i, 0)),
            _full((1, _C)), _full((1, _C)),
            _full((_C, _A)), _full((1, _A)),
            _full((_P, _A)), _full((_P, _C)),
            _full((_C, _C)), _full((1, _C)), _full((_C, _C)), _full((1, _C)),
        ],
        out_specs=[
            pl.BlockSpec((_B4, _C), lambda i: (i, 0)),
            pl.BlockSpec((_B4, 128), lambda i: (i, 0)),
        ],
        out_shape=[
            jax.ShapeDtypeStruct((_N, _C), _f32),
            jax.ShapeDtypeStruct((_N, 128), jnp.int32),
        ],
    )(he, al_full, ar_full, lg2, lb2, wst, bs, d, g,
      wat, ba, wot, bo)

    cluster_mapping = ci[:, 0]
    loss = jnp.float32(0.0)
    return y, loss, reps, cluster_mapping
